# Initial kernel scaffold; baseline (speedup 1.0000x reference)
import jax
import jax.numpy as jnp
from jax.experimental import pallas as pl


def kernel(atom_number, embedding_list):
    raise NotImplementedError("write your pallas kernel here")


# speedup vs baseline: 1.7343x; 1.7343x over previous
import functools

import jax
import jax.numpy as jnp
from jax import lax
from jax.experimental import pallas as pl
from jax.experimental.pallas import tpu as pltpu
from jax.experimental.pallas import tpu_sc as plsc

_INFO = plsc.get_sparse_core_info()
_NC, _NS = _INFO.num_cores, _INFO.num_subcores
_NW = _NC * _NS

_B = 16384 * 50
_D = 128
_CHUNK = 128
_PER_W = _B // _NW
_ITERS = _PER_W // _CHUNK


def _make_lookup():
    mesh = plsc.VectorSubcoreMesh(core_axis_name="c", subcore_axis_name="s")

    @functools.partial(
        pl.kernel,
        mesh=mesh,
        out_type=jax.ShapeDtypeStruct((_B, _D), jnp.float32),
        scratch_types=[
            pltpu.VMEM((_CHUNK,), jnp.int32),
            pltpu.VMEM((_CHUNK, _D), jnp.float32),
            pltpu.SemaphoreType.DMA,
        ],
    )
    def lookup(table_hbm, idx_hbm, out_hbm, idx_v, rows_v, sem):
        wid = lax.axis_index("s") * _NC + lax.axis_index("c")
        base = wid * _PER_W

        def body(i, carry):
            off = base + i * _CHUNK
            pltpu.sync_copy(idx_hbm.at[pl.ds(off, _CHUNK)], idx_v)
            pltpu.async_copy(table_hbm.at[idx_v], rows_v, sem).wait()
            pltpu.sync_copy(rows_v, out_hbm.at[pl.ds(off, _CHUNK)])
            return carry

        lax.fori_loop(0, _ITERS, body, 0)

    return lookup


_lookup = _make_lookup()


def kernel(atom_number, embedding_list):
    idx = atom_number.reshape(_B)
    out = _lookup(embedding_list, idx)
    return out.reshape(atom_number.shape[0], atom_number.shape[1], _D)


# speedup vs baseline: 1.7410x; 1.0038x over previous
import functools

import jax
import jax.numpy as jnp
from jax import lax
from jax.experimental import pallas as pl
from jax.experimental.pallas import tpu as pltpu
from jax.experimental.pallas import tpu_sc as plsc

_INFO = plsc.get_sparse_core_info()
_NC, _NS = _INFO.num_cores, _INFO.num_subcores
_NW = _NC * _NS

_B = 16384 * 50
_D = 128
_CHUNK = 128
_PER_W = _B // _NW
_ITERS = _PER_W // _CHUNK
_IDX_ROWS = _ITERS


def _make_lookup():
    mesh = plsc.VectorSubcoreMesh(core_axis_name="c", subcore_axis_name="s")

    @functools.partial(
        pl.kernel,
        mesh=mesh,
        out_type=jax.ShapeDtypeStruct((_B, _D), jnp.float32),
        scratch_types=[
            pltpu.VMEM((_IDX_ROWS, _CHUNK), jnp.int32),
            pltpu.VMEM((_CHUNK, _D), jnp.float32),
            pltpu.VMEM((_CHUNK, _D), jnp.float32),
            pltpu.SemaphoreType.DMA,
            pltpu.SemaphoreType.DMA,
        ],
    )
    def lookup(table_hbm, idx_hbm, out_hbm, idx_v, rows0, rows1, gsem, wsem):
        wid = lax.axis_index("s") * _NC + lax.axis_index("c")
        base = wid * _PER_W
        pltpu.sync_copy(idx_hbm.at[pl.ds(wid * _IDX_ROWS, _IDX_ROWS)], idx_v)
        bufs = (rows0, rows1)

        pltpu.async_copy(table_hbm.at[idx_v.at[0]], rows0, gsem)

        def outer(c2, carry):
            for p in range(2):
                c = c2 * 2 + p
                buf = bufs[p]
                nxt = bufs[1 - p]

                @pl.when(c >= 1)
                def _wait_prev_write():
                    pltpu.make_async_copy(
                        nxt, out_hbm.at[pl.ds(base + (c - 1) * _CHUNK, _CHUNK)], wsem
                    ).wait()

                @pl.when(c + 1 < _ITERS)
                def _issue_next_gather():
                    pltpu.async_copy(table_hbm.at[idx_v.at[c + 1]], nxt, gsem)

                pltpu.make_async_copy(table_hbm.at[idx_v.at[c]], buf, gsem).wait()
                pltpu.async_copy(
                    buf, out_hbm.at[pl.ds(base + c * _CHUNK, _CHUNK)], wsem
                )
            return carry

        lax.fori_loop(0, _ITERS // 2, outer, 0)
        pltpu.make_async_copy(
            bufs[1], out_hbm.at[pl.ds(base + (_ITERS - 1) * _CHUNK, _CHUNK)], wsem
        ).wait()

    return lookup


_lookup = _make_lookup()


def kernel(atom_number, embedding_list):
    idx = atom_number.reshape(_B // _CHUNK, _CHUNK)
    out = _lookup(embedding_list, idx)
    return out.reshape(atom_number.shape[0], atom_number.shape[1], _D)


# speedup vs baseline: 3.2112x; 1.8445x over previous
import functools

import jax
import jax.numpy as jnp
from jax import lax
from jax.experimental import pallas as pl
from jax.experimental.pallas import tpu as pltpu
from jax.experimental.pallas import tpu_sc as plsc

_INFO = plsc.get_sparse_core_info()
_NC, _NS = _INFO.num_cores, _INFO.num_subcores
_NW = _NC * _NS

_B = 16384 * 50
_D = 128
_V = 83
_CHUNK = 128
_PER_W = _B // _NW
_ITERS = _PER_W // _CHUNK
_IDX_ROWS = _ITERS


def _make_lookup():
    mesh = plsc.VectorSubcoreMesh(core_axis_name="c", subcore_axis_name="s")

    @functools.partial(
        pl.kernel,
        mesh=mesh,
        out_type=jax.ShapeDtypeStruct((_B, _D), jnp.float32),
        scratch_types=[
            pltpu.VMEM((_IDX_ROWS, _CHUNK), jnp.int32),
            pltpu.VMEM((_CHUNK, _D), jnp.float32),
            pltpu.VMEM((_CHUNK, _D), jnp.float32),
            pltpu.SemaphoreType.DMA,
            pltpu.SemaphoreType.DMA,
        ],
    )
    def lookup(table_hbm, idx_hbm, out_hbm, idx_v, rows0, rows1, gsem, wsem):
        wid = lax.axis_index("s") * _NC + lax.axis_index("c")
        base = wid * _PER_W
        pltpu.sync_copy(idx_hbm.at[pl.ds(wid * _IDX_ROWS, _IDX_ROWS)], idx_v)
        bufs = (rows0, rows1)

        pltpu.async_copy(table_hbm.at[idx_v.at[0]], rows0, gsem)

        def outer(c2, carry):
            for p in range(2):
                c = c2 * 2 + p
                buf = bufs[p]
                nxt = bufs[1 - p]

                @pl.when(c >= 1)
                def _wait_prev_write():
                    pltpu.make_async_copy(
                        nxt, out_hbm.at[pl.ds(base + (c - 1) * _CHUNK, _CHUNK)], wsem
                    ).wait()

                @pl.when(c + 1 < _ITERS)
                def _issue_next_gather():
                    pltpu.async_copy(table_hbm.at[idx_v.at[c + 1]], nxt, gsem)

                pltpu.make_async_copy(table_hbm.at[idx_v.at[c]], buf, gsem).wait()
                pltpu.async_copy(
                    buf, out_hbm.at[pl.ds(base + c * _CHUNK, _CHUNK)], wsem
                )
            return carry

        lax.fori_loop(0, _ITERS // 2, outer, 0)
        pltpu.make_async_copy(
            bufs[1], out_hbm.at[pl.ds(base + (_ITERS - 1) * _CHUNK, _CHUNK)], wsem
        ).wait()

    return lookup


_lookup = _make_lookup()


def kernel(atom_number, embedding_list):
    table_rep = jnp.tile(embedding_list, (_NW, 1))
    idx = atom_number.reshape(_B // _CHUNK, _CHUNK)
    woff = jnp.repeat(
        jnp.arange(_NW, dtype=jnp.int32) * _V, _B // _CHUNK // _NW
    )[:, None]
    out = _lookup(table_rep, idx + woff)
    return out.reshape(atom_number.shape[0], atom_number.shape[1], _D)
